# baseline (speedup 1.0000x reference)
import functools

import jax
import jax.numpy as jnp
from jax import lax
from jax.experimental import pallas as pl
from jax.experimental.pallas import tpu as pltpu
from jax.experimental.pallas import tpu_sc as plsc

N = 10000
E = 320000
NPAD = 10240
STG = 2000


def _mm3_body(x_ref, w_ref, o1_ref, o2_ref, o3_ref, *, do):
    acc = jnp.dot(x_ref[...], w_ref[...], preferred_element_type=jnp.float32)
    o1_ref[...] = acc[:, :do]
    o2_ref[...] = acc[:, do:2 * do]
    o3_ref[...] = acc[:, 2 * do:]


def _mm3(h, wcat):
    n, k = h.shape
    do = wcat.shape[1] // 3
    bn = 2000
    grid = (n // bn,)
    osp = jax.ShapeDtypeStruct((n, do), jnp.float32)
    xl, xr, lin = pl.pallas_call(
        functools.partial(_mm3_body, do=do),
        grid=grid,
        in_specs=[
            pl.BlockSpec((bn, k), lambda i: (i, 0)),
            pl.BlockSpec((k, 3 * do), lambda i: (0, 0)),
        ],
        out_specs=[pl.BlockSpec((bn, do), lambda i: (i, 0))] * 3,
        out_shape=[osp, osp, osp],
    )(h, wcat)
    return xl, xr, lin


def _epi_body(num_ref, den_ref, lin_ref, bias_ref, o_ref, *, relu):
    den = den_ref[:, 0:1] + 1e-16
    h = num_ref[...] / den + lin_ref[...] + bias_ref[...]
    if relu:
        h = jnp.maximum(h, 0.0)
    o_ref[...] = h


def _epilogue(num, den, lin, bias, relu):
    n, do = lin.shape
    bn = 2000
    grid = (n // bn,)
    return pl.pallas_call(
        functools.partial(_epi_body, relu=relu),
        grid=grid,
        in_specs=[
            pl.BlockSpec((bn, do), lambda i: (i, 0)),
            pl.BlockSpec((bn, 16), lambda i: (i, 0)),
            pl.BlockSpec((bn, do), lambda i: (i, 0)),
            pl.BlockSpec((1, do), lambda i: (0, 0)),
        ],
        out_specs=pl.BlockSpec((bn, do), lambda i: (i, 0)),
        out_shape=jax.ShapeDtypeStruct((n, do), jnp.float32),
    )(num, den, lin, bias)


def _make_edge_kernel(do, nhalf, half):
    nkc = do // 16
    rows_pt = NPAD // (32 * nhalf)
    arows = rows_pt + 8
    nstg = E // STG
    nrows = NPAD // nhalf
    lcap = STG + 32

    mesh = plsc.VectorSubcoreMesh(core_axis_name="c", subcore_axis_name="s")

    out_types = [jax.ShapeDtypeStruct((nrows * do,), jnp.float32),
                 jax.ShapeDtypeStruct((nrows * 16,), jnp.float32)]

    scratch = [
        pltpu.VMEM((STG,), jnp.int32),
        pltpu.VMEM((STG,), jnp.int32),
        pltpu.VMEM((lcap,), jnp.int32),
        pltpu.VMEM((lcap,), jnp.int32),
        pltpu.VMEM((16,), jnp.int32),
        pltpu.VMEM((16,), jnp.int32),
        pltpu.VMEM((arows * 16,), jnp.float32),
        pltpu.VMEM((16, do), jnp.float32),
        pltpu.VMEM((16, do), jnp.float32),
        pltpu.VMEM((arows * do,), jnp.float32),
        pltpu.VMEM((do,), jnp.float32),
        pltpu.SemaphoreType.DMA,
    ]

    @functools.partial(
        pl.kernel, mesh=mesh, out_type=out_types, scratch_types=scratch,
    )
    def edge_kernel(xl_hbm, xr_hbm, src_hbm, dst_hbm, att_hbm,
                    out_hbm, den_hbm,
                    srcbuf, dstbuf, srclist, dstlist, gidx, sidx,
                    accden, xlb, xrb, accf, attbuf, gsem):
        cid = lax.axis_index("c")
        sid = lax.axis_index("s")
        wid = sid * 2 + cid
        row_base = half * nrows + wid * rows_pt
        hbase = wid * rows_pt

        pltpu.sync_copy(att_hbm, attbuf)

        def zbody(i, c):
            accf[pl.ds(i * 16, 16)] = jnp.zeros((16,), jnp.float32)
            return c
        lax.fori_loop(0, arows * do // 16, zbody, jnp.int32(0))

        def zdbody(i, c):
            accden[pl.ds(i * 16, 16)] = jnp.zeros((16,), jnp.float32)
            return c
        lax.fori_loop(0, arows, zdbody, jnp.int32(0))

        def _append(listref, cn, val):
            base = pl.multiple_of((cn // 8) * 8, 8)
            lane = lax.iota(jnp.int32, 16)
            w = listref[pl.ds(base, 16)]
            listref[pl.ds(base, 16)] = jnp.where(
                lane == cn - base, jnp.full((16,), val, jnp.int32), w)

        def stage_body(st, c0):
            sbase = st * STG
            pltpu.sync_copy(src_hbm.at[pl.ds(sbase, STG)], srcbuf)
            pltpu.sync_copy(dst_hbm.at[pl.ds(sbase, STG)], dstbuf)

            def scan_body(j, cn):
                dv = dstbuf[pl.ds(j * 16, 16)]
                sv = srcbuf[pl.ds(j * 16, 16)]
                cn2 = cn
                for l in range(16):
                    dvl = dv[l]
                    svl = sv[l]
                    lvl = dvl - row_base
                    ml = ((lvl >= 0) & (lvl < rows_pt)).astype(jnp.int32)
                    _append(srclist, cn2, svl)
                    _append(dstlist, cn2, dvl)
                    cn2 = cn2 + ml
                return cn2

            cnt = lax.fori_loop(0, STG // 16, scan_body, jnp.int32(0))

            for _ in range(16):
                _append(srclist, cnt, jnp.int32(0))
                _append(dstlist, cnt, jnp.int32(-1))
                cnt = cnt + 1
            nch = (cnt // 16) - 1 + jnp.minimum(cnt % 16, 1)

            def chunk_body(gi, c1):
                goff = gi * 16
                dvv = dstlist[pl.ds(goff, 16)]
                svv = srclist[pl.ds(goff, 16)]
                ok = dvv >= 0
                sidx[...] = jnp.where(ok, svv, jnp.zeros((16,), jnp.int32))
                gidx[...] = jnp.where(ok, dvv, jnp.zeros((16,), jnp.int32))
                cp1 = pltpu.async_copy(xl_hbm.at[sidx], xlb, gsem)
                cp2 = pltpu.async_copy(xr_hbm.at[gidx], xrb, gsem)
                cp1.wait()
                cp2.wait()
                lvv = jnp.where(ok, dvv - row_base,
                                jnp.full((16,), rows_pt, jnp.int32))
                for e in range(16):
                    acc16 = jnp.zeros((16,), jnp.float32)

                    def dot_body(k, a, e=e):
                        av = xlb[e, pl.ds(k * 16, 16)]
                        rv = xrb[e, pl.ds(k * 16, 16)]
                        t = av + rv
                        t = jnp.maximum(t, t * 0.2)
                        return a + t * attbuf[pl.ds(k * 16, 16)]

                    acc16 = lax.fori_loop(0, nkc, dot_body, acc16)
                    s0 = (((acc16[0] + acc16[1]) + (acc16[2] + acc16[3]))
                          + ((acc16[4] + acc16[5]) + (acc16[6] + acc16[7]))
                          + ((acc16[8] + acc16[9]) + (acc16[10] + acc16[11]))
                          + ((acc16[12] + acc16[13])
                             + (acc16[14] + acc16[15])))
                    exb = jnp.exp(jnp.full((16,), s0, jnp.float32))
                    lvs = lvv[e]

                    def acc_body(k, c4, e=e, lvs=lvs, exb=exb):
                        base = lvs * do + k * 16
                        cur = accf[pl.ds(base, 16)]
                        accf[pl.ds(base, 16)] = (
                            cur + exb * xlb[e, pl.ds(k * 16, 16)])
                        return c4

                    lax.fori_loop(0, nkc, acc_body, jnp.int32(0))
                    lane2 = lax.iota(jnp.int32, 16)
                    oh = jnp.where(lane2 == 0, exb,
                                   jnp.zeros((16,), jnp.float32))
                    dbase = lvs * 16
                    accden[pl.ds(dbase, 16)] = (
                        accden[pl.ds(dbase, 16)] + oh)
                return c1

            lax.fori_loop(0, nch, chunk_body, jnp.int32(0))
            return c0

        lax.fori_loop(0, nstg, stage_body, jnp.int32(0))

        pltpu.sync_copy(
            accf.at[pl.ds(0, rows_pt * do)],
            out_hbm.at[pl.ds(hbase * do, rows_pt * do)])
        pltpu.sync_copy(
            accden.at[pl.ds(0, rows_pt * 16)],
            den_hbm.at[pl.ds(hbase * 16, rows_pt * 16)])

    return edge_kernel


_EDGE_KERNELS = {}


def _edge_pass(xl, xr, src, dst, att):
    do = xl.shape[1]
    nhalf = 2 if do == 512 else 1
    nrows = NPAD // nhalf
    half_outs = []
    for half in range(nhalf):
        cfg = (do, nhalf, half)
        if cfg not in _EDGE_KERNELS:
            _EDGE_KERNELS[cfg] = _make_edge_kernel(do, nhalf, half)
        kfn = _EDGE_KERNELS[cfg]
        num_f, den_f = kfn(xl, xr, src, dst, att)
        half_outs.append((num_f.reshape(nrows, do), den_f.reshape(nrows, 16)))
    if nhalf == 1:
        num = half_outs[0][0][:N]
        den = half_outs[0][1][:N]
    else:
        num = jnp.concatenate([h[0] for h in half_outs])[:N]
        den = jnp.concatenate([h[1] for h in half_outs])[:N]
    return num, den


def kernel(x, edge_index, Wl1, Wr1, att1, b1, Wlin1, blin1,
           Wl2, Wr2, att2, b2, Wlin2, blin2,
           Wl3, Wr3, att3, b3, Wlin3, blin3):
    src = edge_index[0]
    dst = edge_index[1]
    h = x
    layers = [
        (Wl1, Wr1, att1, b1, Wlin1, blin1, True),
        (Wl2, Wr2, att2, b2, Wlin2, blin2, True),
        (Wl3, Wr3, att3, b3, Wlin3, blin3, False),
    ]
    for Wl, Wr, att, b, Wlin, blin, relu in layers:
        wcat = jnp.concatenate([Wl, Wr, Wlin], axis=1)
        xl, xr, lin = _mm3(h, wcat)
        num, den = _edge_pass(xl, xr, src, dst, att)
        bias = (b + blin).reshape(1, -1)
        h = _epilogue(num, den, lin, bias, relu)
    return h
